# baseline (speedup 1.0000x reference)
import jax
import jax.numpy as jnp
from jax import lax
from jax.experimental import pallas as pl
from jax.experimental.pallas import tpu as pltpu
from jax.experimental.pallas import tpu_sc as plsc

N_STARS = 1000000
BATCH = 16384
N_FREQ = 200
EPS = 0.001

_NC = 2
_NS = 16
_NW = _NC * _NS
_BPW = BATCH // _NW
_L = 16


def _make_delta_gather_body(bpw):
    def _delta_gather_body(idx_hbm, hard_hbm, corr_hbm, out_hbm, idx_v, h_v, c_v, sem):
        wid = lax.axis_index("s") * _NC + lax.axis_index("c")
        base = wid * bpw
        pltpu.sync_copy(idx_hbm.at[pl.ds(base, bpw)], idx_v)
        cp_h = pltpu.async_copy(hard_hbm.at[idx_v], h_v, sem)
        cp_c = pltpu.async_copy(corr_hbm.at[idx_v], c_v, sem)
        cp_h.wait()
        cp_c.wait()
        eps = jnp.float32(EPS)
        for i in range(bpw // _L):
            s = pl.ds(i * _L, _L)
            h_v[s] = jnp.maximum(h_v[s] + c_v[s], eps)
        pltpu.sync_copy(h_v, out_hbm.at[pl.ds(base, bpw)])
    return _delta_gather_body


def _gather_delta(star_indices, hard, corr):
    n = star_indices.shape[0]
    bpw = n // _NW
    mesh = plsc.VectorSubcoreMesh(core_axis_name="c", subcore_axis_name="s")
    fn = pl.kernel(
        _make_delta_gather_body(bpw),
        mesh=mesh,
        out_type=jax.ShapeDtypeStruct((n,), jnp.float32),
        scratch_types=[
            pltpu.VMEM((bpw,), jnp.int32),
            pltpu.VMEM((bpw,), jnp.float32),
            pltpu.VMEM((bpw,), jnp.float32),
            pltpu.SemaphoreType.DMA,
        ],
    )
    return fn(star_indices, hard, corr)


_BS = 40
_GT = -(-N_FREQ // _BS)


def _mod_body2(d0_ref, d1_ref, f_ref, o_ref):
    d = jnp.concatenate([d0_ref[...], d1_ref[...]])
    o_ref[...] = jnp.mod(f_ref[...], d[None, :])


_H = BATCH // 2


def _apply_mod_t2(freq_t, d0, d1):
    return pl.pallas_call(
        _mod_body2,
        grid=(_GT,),
        in_specs=[
            pl.BlockSpec((_H,), lambda i: (0,)),
            pl.BlockSpec((_H,), lambda i: (0,)),
            pl.BlockSpec((_BS, BATCH), lambda i: (i, 0)),
        ],
        out_specs=pl.BlockSpec((_BS, BATCH), lambda i: (i, 0)),
        out_shape=jax.ShapeDtypeStruct((N_FREQ, BATCH), jnp.float32),
    )(d0, d1, freq_t)


def kernel(frequencies, star_indices, delta_nu_hard, delta_nu_corr):
    idx = star_indices.astype(jnp.int32)
    d0 = _gather_delta(idx[:_H], delta_nu_hard, delta_nu_corr)
    d1 = _gather_delta(idx[_H:], delta_nu_hard, delta_nu_corr)
    out_t = _apply_mod_t2(frequencies.T, d0, d1)
    return out_t.T


# speedup vs baseline: 1.1444x; 1.1444x over previous
import jax
import jax.numpy as jnp
from jax import lax
from jax.experimental import pallas as pl
from jax.experimental.pallas import tpu as pltpu
from jax.experimental.pallas import tpu_sc as plsc

N_STARS = 1000000
BATCH = 16384
N_FREQ = 200
EPS = 0.001

_NC = 2
_NS = 16
_NW = _NC * _NS
_BPW = BATCH // _NW
_L = 16


def _delta_gather_body(idx_hbm, hard_hbm, corr_hbm, out_hbm, idx_v, h_v, c_v, sem):
    wid = lax.axis_index("s") * _NC + lax.axis_index("c")
    base = wid * _BPW
    pltpu.sync_copy(idx_hbm.at[pl.ds(base, _BPW)], idx_v)
    cp_h = pltpu.async_copy(hard_hbm.at[idx_v], h_v, sem)
    cp_c = pltpu.async_copy(corr_hbm.at[idx_v], c_v, sem)
    cp_h.wait()
    cp_c.wait()
    eps = jnp.float32(EPS)
    for i in range(_BPW // _L):
        s = pl.ds(i * _L, _L)
        h_v[s] = jnp.maximum(h_v[s] + c_v[s], eps)
    pltpu.sync_copy(h_v, out_hbm.at[pl.ds(base, _BPW)])


def _gather_delta(star_indices, hard, corr):
    mesh = plsc.VectorSubcoreMesh(core_axis_name="c", subcore_axis_name="s")
    fn = pl.kernel(
        _delta_gather_body,
        mesh=mesh,
        out_type=jax.ShapeDtypeStruct((BATCH,), jnp.float32),
        scratch_types=[
            pltpu.VMEM((_BPW,), jnp.int32),
            pltpu.VMEM((_BPW,), jnp.float32),
            pltpu.VMEM((_BPW,), jnp.float32),
            pltpu.SemaphoreType.DMA,
        ],
    )
    return fn(star_indices, hard, corr)


_BS = 40
_GT = -(-N_FREQ // _BS)


def _mod_body(d_ref, f_ref, o_ref):
    o_ref[...] = jnp.mod(f_ref[...], d_ref[...][None, :])


def _apply_mod_t(freq_t, delta):
    return pl.pallas_call(
        _mod_body,
        grid=(_GT,),
        in_specs=[
            pl.BlockSpec((BATCH,), lambda i: (0,)),
            pl.BlockSpec((_BS, BATCH), lambda i: (i, 0)),
        ],
        out_specs=pl.BlockSpec((_BS, BATCH), lambda i: (i, 0)),
        out_shape=jax.ShapeDtypeStruct((N_FREQ, BATCH), jnp.float32),
    )(delta, freq_t)


def kernel(frequencies, star_indices, delta_nu_hard, delta_nu_corr):
    idx = star_indices.astype(jnp.int32)
    delta = _gather_delta(idx, delta_nu_hard, delta_nu_corr)
    out_t = _apply_mod_t(frequencies.T, delta)
    return out_t.T


# speedup vs baseline: 1.1615x; 1.0149x over previous
import jax
import jax.numpy as jnp
from jax import lax
from jax.experimental import pallas as pl
from jax.experimental.pallas import tpu as pltpu
from jax.experimental.pallas import tpu_sc as plsc

N_STARS = 1000000
BATCH = 16384
N_FREQ = 200
EPS = 0.001

_NC = 2
_NS = 16
_NW = _NC * _NS
_BPW = BATCH // _NW
_L = 16


def _delta_gather_body(idx_hbm, hard_hbm, corr_hbm, out_hbm, idx_v, h_v, c_v, sem):
    wid = lax.axis_index("s") * _NC + lax.axis_index("c")
    base = wid * _BPW
    pltpu.sync_copy(idx_hbm.at[pl.ds(base, _BPW)], idx_v)
    cp_h = pltpu.async_copy(hard_hbm.at[idx_v], h_v, sem)
    cp_c = pltpu.async_copy(corr_hbm.at[idx_v], c_v, sem)
    cp_h.wait()
    cp_c.wait()
    eps = jnp.float32(EPS)
    for i in range(_BPW // _L):
        s = pl.ds(i * _L, _L)
        h_v[s] = jnp.maximum(h_v[s] + c_v[s], eps)
    pltpu.sync_copy(h_v, out_hbm.at[pl.ds(base, _BPW)])


def _gather_delta(star_indices, hard, corr):
    mesh = plsc.VectorSubcoreMesh(core_axis_name="c", subcore_axis_name="s")
    fn = pl.kernel(
        _delta_gather_body,
        mesh=mesh,
        out_type=jax.ShapeDtypeStruct((BATCH,), jnp.float32),
        scratch_types=[
            pltpu.VMEM((_BPW,), jnp.int32),
            pltpu.VMEM((_BPW,), jnp.float32),
            pltpu.VMEM((_BPW,), jnp.float32),
            pltpu.SemaphoreType.DMA,
        ],
    )
    return fn(star_indices, hard, corr)


_BS = 48
_GT = -(-N_FREQ // _BS)


def _mod_body(d_ref, f_ref, o_ref):
    o_ref[...] = jnp.mod(f_ref[...], d_ref[...][None, :])


def _apply_mod_t(freq_t, delta):
    return pl.pallas_call(
        _mod_body,
        grid=(_GT,),
        in_specs=[
            pl.BlockSpec((BATCH,), lambda i: (0,)),
            pl.BlockSpec((_BS, BATCH), lambda i: (i, 0)),
        ],
        out_specs=pl.BlockSpec((_BS, BATCH), lambda i: (i, 0)),
        out_shape=jax.ShapeDtypeStruct((N_FREQ, BATCH), jnp.float32),
    )(delta, freq_t)


def kernel(frequencies, star_indices, delta_nu_hard, delta_nu_corr):
    idx = star_indices.astype(jnp.int32)
    delta = _gather_delta(idx, delta_nu_hard, delta_nu_corr)
    out_t = _apply_mod_t(frequencies.T, delta)
    return out_t.T


# speedup vs baseline: 1.1688x; 1.0063x over previous
import jax
import jax.numpy as jnp
from jax import lax
from jax.experimental import pallas as pl
from jax.experimental.pallas import tpu as pltpu
from jax.experimental.pallas import tpu_sc as plsc

N_STARS = 1000000
BATCH = 16384
N_FREQ = 200
EPS = 0.001

_NC = 2
_NS = 16
_NW = _NC * _NS
_BPW = BATCH // _NW
_L = 16


def _delta_gather_body(idx_hbm, hard_hbm, corr_hbm, out_hbm, idx_v, h_v, c_v, sem):
    wid = lax.axis_index("s") * _NC + lax.axis_index("c")
    base = wid * _BPW
    pltpu.sync_copy(idx_hbm.at[pl.ds(base, _BPW)], idx_v)
    cp_h = pltpu.async_copy(hard_hbm.at[idx_v], h_v, sem)
    cp_c = pltpu.async_copy(corr_hbm.at[idx_v], c_v, sem)
    cp_h.wait()
    cp_c.wait()
    eps = jnp.float32(EPS)
    for i in range(_BPW // _L):
        s = pl.ds(i * _L, _L)
        h_v[s] = jnp.maximum(h_v[s] + c_v[s], eps)
    pltpu.sync_copy(h_v, out_hbm.at[pl.ds(base, _BPW)])


def _gather_delta(star_indices, hard, corr):
    mesh = plsc.VectorSubcoreMesh(core_axis_name="c", subcore_axis_name="s")
    fn = pl.kernel(
        _delta_gather_body,
        mesh=mesh,
        out_type=jax.ShapeDtypeStruct((BATCH,), jnp.float32),
        scratch_types=[
            pltpu.VMEM((_BPW,), jnp.int32),
            pltpu.VMEM((_BPW,), jnp.float32),
            pltpu.VMEM((_BPW,), jnp.float32),
            pltpu.SemaphoreType.DMA,
        ],
    )
    return fn(star_indices, hard, corr)


_BS = 56
_GT = -(-N_FREQ // _BS)


def _mod_body(d_ref, f_ref, o_ref):
    o_ref[...] = jnp.mod(f_ref[...], d_ref[...][None, :])


def _apply_mod_t(freq_t, delta):
    return pl.pallas_call(
        _mod_body,
        grid=(_GT,),
        in_specs=[
            pl.BlockSpec((BATCH,), lambda i: (0,)),
            pl.BlockSpec((_BS, BATCH), lambda i: (i, 0)),
        ],
        out_specs=pl.BlockSpec((_BS, BATCH), lambda i: (i, 0)),
        out_shape=jax.ShapeDtypeStruct((N_FREQ, BATCH), jnp.float32),
    )(delta, freq_t)


def kernel(frequencies, star_indices, delta_nu_hard, delta_nu_corr):
    idx = star_indices.astype(jnp.int32)
    delta = _gather_delta(idx, delta_nu_hard, delta_nu_corr)
    out_t = _apply_mod_t(frequencies.T, delta)
    return out_t.T
